# baseline (speedup 1.0000x reference)
import functools
import math

import jax
import jax.numpy as jnp
from jax import lax
from jax.experimental import pallas as pl
from jax.experimental.pallas import tpu as pltpu
from jax.experimental.pallas import tpu_sc as plsc

_N = 8192
_D = 768
_K = 512
_BN = 1024
_SQ2OPI = math.sqrt(2.0 / math.pi)

_NC = 2
_NS = 16
_NW = _NC * _NS
_RPT = _N // _NW
_CH = 64
_KPT = _K // _NS


def _row_normalize(v, eps):
    n = jnp.sqrt(jnp.sum(v * v, axis=-1, keepdims=True))
    return v / jnp.maximum(n, eps)


def _assign_kernel(x_ref, p_ref, assign_ref):
    xb = x_ref[...]
    p_norm = _row_normalize(p_ref[...], 1e-12)
    sims = jax.lax.dot_general(xb, p_norm, (((1,), (1,)), ((), ())),
                               preferred_element_type=jnp.float32)
    assign_ref[...] = jnp.argmax(sims, axis=-1).astype(jnp.int32).reshape(1, 1, _BN)


_sc_mesh = plsc.VectorSubcoreMesh(core_axis_name="c", subcore_axis_name="s")

_NSTRIPE = _D // 128
_NQ = 4
_NACT = _NSTRIPE * _NQ
_QROWS = _N // _NQ
_CH2 = 256


@functools.partial(
    pl.kernel,
    mesh=_sc_mesh,
    out_type=jax.ShapeDtypeStruct((_NQ, _K, _D), jnp.float32),
    scratch_types=[
        pltpu.VMEM((_CH2,), jnp.int32),
        pltpu.VMEM((_CH2, 128), jnp.float32),
        pltpu.VMEM((_K, 128), jnp.float32),
    ],
    compiler_params=pltpu.CompilerParams(needs_layout_passes=False),
)
def _segsum_sc(x_hbm, a_hbm, z_hbm, sums_out, idx_v, rows_v, acc_v):
    cid = lax.axis_index("c")
    sid = lax.axis_index("s")
    wid = sid * _NC + cid
    lanes = lax.iota(jnp.int32, 16)

    @pl.when(wid < _NACT)
    def _active():
        stripe = wid // _NQ
        rowq = wid % _NQ
        c0 = stripe * 128
        pltpu.sync_copy(z_hbm, acc_v)

        def row_body(r, carry):
            rsplat = jnp.broadcast_to(r, (16,)).astype(jnp.int32)
            a_splat = plsc.load_gather(idx_v, [rsplat])
            for j in range(8):
                cols = lanes + (16 * j)
                vals = plsc.load_gather(rows_v, [rsplat, cols])
                plsc.addupdate_scatter(acc_v, [a_splat, cols], vals)
            return carry

        for ci in range(_QROWS // _CH2):
            off = rowq * _QROWS + ci * _CH2
            pltpu.sync_copy(a_hbm.at[pl.ds(off, _CH2)], idx_v)
            pltpu.sync_copy(x_hbm.at[pl.ds(off, _CH2), pl.ds(c0, 128)], rows_v)
            lax.fori_loop(0, _CH2, row_body, 0)

        pltpu.sync_copy(acc_v, sums_out.at[rowq, :, pl.ds(c0, 128)])


def _update_kernel(p_ref, sums_ref, assign_ref, pn2_ref):
    p0 = p_ref[...]
    sums = sums_ref[0] + sums_ref[1] + sums_ref[2] + sums_ref[3]
    a = assign_ref[...].reshape(1, _N)
    onehot = (jax.lax.broadcasted_iota(jnp.int32, (_K, _N), 0) == a)
    counts = jnp.sum(onehot.astype(jnp.float32), axis=1).reshape(_K, 1)
    centroids = jnp.where(counts > 0, sums / jnp.maximum(counts, 1.0), p0)
    new_p = _row_normalize(centroids, 1e-12)
    p_upd = 0.999 * p0 + 0.001 * new_p
    pn2_ref[...] = _row_normalize(p_upd, 1e-08)


def _out_kernel(x_ref, pn2_ref, lt_ref, lb_ref, out_ref):
    xb = x_ref[...]
    pn2 = pn2_ref[...]
    s2 = jax.lax.dot_general(xb, pn2, (((1,), (1,)), ((), ())),
                             preferred_element_type=jnp.float32)
    rowmax = jnp.max(s2, axis=-1)
    xnorm = jnp.sqrt(jnp.sum(xb * xb, axis=-1))
    m = rowmax / jnp.maximum(xnorm, 1e-08)
    m = jnp.clip(m, -1.0, 1.0)
    dists = jnp.clip(1.0 - m, 0.0, 2.0)
    tau = jnp.exp(lt_ref[0, 0])
    alpha = jax.nn.sigmoid(lb_ref[0, 0])
    novelty = 1.0 - jnp.exp(-tau * dists)
    scale = jnp.clip(1.0 - alpha + alpha * novelty, 0.1, 10.0)[:, None]
    y = xb * scale
    out_ref[...] = 0.5 * y * (1.0 + jnp.tanh(_SQ2OPI * (y + 0.044715 * y**3)))


@jax.jit
def _run(x2d, P, log_tau, log_blend):
    nblk = _N // _BN
    assign3 = pl.pallas_call(
        _assign_kernel,
        grid=(nblk,),
        in_specs=[
            pl.BlockSpec((_BN, _D), lambda i: (i, 0)),
            pl.BlockSpec((_K, _D), lambda i: (0, 0)),
        ],
        out_specs=pl.BlockSpec((1, 1, _BN), lambda i: (i, 0, 0)),
        out_shape=jax.ShapeDtypeStruct((nblk, 1, _BN), jnp.int32),
    )(x2d, P)
    assign = assign3.reshape(_N)

    zsum = jnp.zeros((_K, 128), jnp.float32)
    sums_part = _segsum_sc(x2d, assign, zsum)

    pn2 = pl.pallas_call(
        _update_kernel,
        in_specs=[
            pl.BlockSpec((_K, _D), lambda: (0, 0)),
            pl.BlockSpec((_NQ, _K, _D), lambda: (0, 0, 0)),
            pl.BlockSpec((_N,), lambda: (0,)),
        ],
        out_specs=pl.BlockSpec((_K, _D), lambda: (0, 0)),
        out_shape=jax.ShapeDtypeStruct((_K, _D), jnp.float32),
    )(P, sums_part, assign)

    out2d = pl.pallas_call(
        _out_kernel,
        grid=(nblk,),
        in_specs=[
            pl.BlockSpec((_BN, _D), lambda i: (i, 0)),
            pl.BlockSpec((_K, _D), lambda i: (0, 0)),
            pl.BlockSpec(memory_space=pltpu.SMEM),
            pl.BlockSpec(memory_space=pltpu.SMEM),
        ],
        out_specs=pl.BlockSpec((_BN, _D), lambda i: (i, 0)),
        out_shape=jax.ShapeDtypeStruct((_N, _D), jnp.float32),
    )(x2d, pn2, log_tau, log_blend)
    return out2d


def kernel(x, P, log_tau, log_blend):
    B, T, D = x.shape
    x2d = x.reshape(-1, D)
    lt = jnp.reshape(log_tau, (1, 1))
    lb = jnp.reshape(log_blend, (1, 1))
    out2d = _run(x2d, P, lt, lb)
    return out2d.reshape(B, T, D)


# speedup vs baseline: 1.6165x; 1.6165x over previous
import functools
import math

import jax
import jax.numpy as jnp
from jax import lax
from jax.experimental import pallas as pl
from jax.experimental.pallas import tpu as pltpu
from jax.experimental.pallas import tpu_sc as plsc

_N = 8192
_D = 768
_K = 512
_BN = 1024
_SQ2OPI = math.sqrt(2.0 / math.pi)

_NC = 2
_NS = 16
_NW = _NC * _NS
_RPT = _N // _NW
_CH = 64
_KPT = _K // _NS


def _row_normalize(v, eps):
    n = jnp.sqrt(jnp.sum(v * v, axis=-1, keepdims=True))
    return v / jnp.maximum(n, eps)


def _assign_kernel(x_ref, p_ref, assign_ref):
    xb = x_ref[...]
    p_norm = _row_normalize(p_ref[...], 1e-12)
    sims = jax.lax.dot_general(xb, p_norm, (((1,), (1,)), ((), ())),
                               preferred_element_type=jnp.float32)
    assign_ref[...] = jnp.argmax(sims, axis=-1).astype(jnp.int32).reshape(1, 1, _BN)


_sc_mesh = plsc.VectorSubcoreMesh(core_axis_name="c", subcore_axis_name="s")

_NSTRIPE = _D // 128
_NQ = 4
_NACT = _NSTRIPE * _NQ
_QROWS = _N // _NQ
_CH2 = 128


@functools.partial(
    pl.kernel,
    mesh=_sc_mesh,
    out_type=jax.ShapeDtypeStruct((_NQ, _K, _D), jnp.float32),
    scratch_types=[
        pltpu.VMEM((_CH2,), jnp.int32),
        pltpu.VMEM((_CH2,), jnp.int32),
        pltpu.VMEM((_CH2, 128), jnp.float32),
        pltpu.VMEM((_CH2, 128), jnp.float32),
        pltpu.VMEM((_K, 128), jnp.float32),
        pltpu.SemaphoreType.DMA,
        pltpu.SemaphoreType.DMA,
    ],
    compiler_params=pltpu.CompilerParams(needs_layout_passes=False),
)
def _segsum_sc(x_hbm, a_hbm, z_hbm, sums_out,
               idx0, idx1, rows0, rows1, acc_v, sem0, sem1):
    cid = lax.axis_index("c")
    sid = lax.axis_index("s")
    wid = sid * _NC + cid
    lanes = lax.iota(jnp.int32, 16)
    nch = _QROWS // _CH2

    @pl.when(wid < _NACT)
    def _active():
        stripe = wid // _NQ
        rowq = wid % _NQ
        c0 = stripe * 128
        pltpu.sync_copy(z_hbm, acc_v)
        banks = ((idx0, rows0, sem0), (idx1, rows1, sem1))

        def start(ci, b):
            off = rowq * _QROWS + ci * _CH2
            ib, rb, sb = banks[b]
            ia = pltpu.make_async_copy(a_hbm.at[pl.ds(off, _CH2)], ib, sb)
            ra = pltpu.make_async_copy(
                x_hbm.at[pl.ds(off, _CH2), pl.ds(c0, 128)], rb, sb)
            ia.start()
            ra.start()
            return ia, ra

        pend = start(0, 0)
        for ci in range(nch):
            b = ci % 2
            pend[0].wait()
            pend[1].wait()
            if ci + 1 < nch:
                pend = start(ci + 1, 1 - b)
            idxb, rowsb, _ = banks[b]

            @plsc.parallel_loop(0, _CH2, unroll=4)
            def _rows(r):
                rsplat = jnp.broadcast_to(r, (16,)).astype(jnp.int32)
                a_splat = plsc.load_gather(idxb, [rsplat])
                for j in range(8):
                    cols = lanes + (16 * j)
                    vals = plsc.load_gather(rowsb, [rsplat, cols])
                    plsc.addupdate_scatter(acc_v, [a_splat, cols], vals)

        pltpu.sync_copy(acc_v, sums_out.at[rowq, :, pl.ds(c0, 128)])


def _update_kernel(p_ref, sums_ref, assign_ref, pn2_ref):
    p0 = p_ref[...]
    sums = sums_ref[0] + sums_ref[1] + sums_ref[2] + sums_ref[3]
    a = assign_ref[...].reshape(1, _N)
    onehot = (jax.lax.broadcasted_iota(jnp.int32, (_K, _N), 0) == a)
    counts = jnp.sum(onehot.astype(jnp.float32), axis=1).reshape(_K, 1)
    centroids = jnp.where(counts > 0, sums / jnp.maximum(counts, 1.0), p0)
    new_p = _row_normalize(centroids, 1e-12)
    p_upd = 0.999 * p0 + 0.001 * new_p
    pn2_ref[...] = _row_normalize(p_upd, 1e-08)


def _out_kernel(x_ref, pn2_ref, lt_ref, lb_ref, out_ref):
    xb = x_ref[...]
    pn2 = pn2_ref[...]
    s2 = jax.lax.dot_general(xb, pn2, (((1,), (1,)), ((), ())),
                             preferred_element_type=jnp.float32)
    rowmax = jnp.max(s2, axis=-1)
    xnorm = jnp.sqrt(jnp.sum(xb * xb, axis=-1))
    m = rowmax / jnp.maximum(xnorm, 1e-08)
    m = jnp.clip(m, -1.0, 1.0)
    dists = jnp.clip(1.0 - m, 0.0, 2.0)
    tau = jnp.exp(lt_ref[0, 0])
    alpha = jax.nn.sigmoid(lb_ref[0, 0])
    novelty = 1.0 - jnp.exp(-tau * dists)
    scale = jnp.clip(1.0 - alpha + alpha * novelty, 0.1, 10.0)[:, None]
    y = xb * scale
    out_ref[...] = 0.5 * y * (1.0 + jnp.tanh(_SQ2OPI * (y + 0.044715 * y**3)))


@jax.jit
def _run(x2d, P, log_tau, log_blend):
    nblk = _N // _BN
    assign3 = pl.pallas_call(
        _assign_kernel,
        grid=(nblk,),
        in_specs=[
            pl.BlockSpec((_BN, _D), lambda i: (i, 0)),
            pl.BlockSpec((_K, _D), lambda i: (0, 0)),
        ],
        out_specs=pl.BlockSpec((1, 1, _BN), lambda i: (i, 0, 0)),
        out_shape=jax.ShapeDtypeStruct((nblk, 1, _BN), jnp.int32),
    )(x2d, P)
    assign = assign3.reshape(_N)

    zsum = jnp.zeros((_K, 128), jnp.float32)
    sums_part = _segsum_sc(x2d, assign, zsum)

    pn2 = pl.pallas_call(
        _update_kernel,
        in_specs=[
            pl.BlockSpec((_K, _D), lambda: (0, 0)),
            pl.BlockSpec((_NQ, _K, _D), lambda: (0, 0, 0)),
            pl.BlockSpec((_N,), lambda: (0,)),
        ],
        out_specs=pl.BlockSpec((_K, _D), lambda: (0, 0)),
        out_shape=jax.ShapeDtypeStruct((_K, _D), jnp.float32),
    )(P, sums_part, assign)

    out2d = pl.pallas_call(
        _out_kernel,
        grid=(nblk,),
        in_specs=[
            pl.BlockSpec((_BN, _D), lambda i: (i, 0)),
            pl.BlockSpec((_K, _D), lambda i: (0, 0)),
            pl.BlockSpec(memory_space=pltpu.SMEM),
            pl.BlockSpec(memory_space=pltpu.SMEM),
        ],
        out_specs=pl.BlockSpec((_BN, _D), lambda i: (i, 0)),
        out_shape=jax.ShapeDtypeStruct((_N, _D), jnp.float32),
    )(x2d, pn2, log_tau, log_blend)
    return out2d


def kernel(x, P, log_tau, log_blend):
    B, T, D = x.shape
    x2d = x.reshape(-1, D)
    lt = jnp.reshape(log_tau, (1, 1))
    lb = jnp.reshape(log_blend, (1, 1))
    out2d = _run(x2d, P, lt, lb)
    return out2d.reshape(B, T, D)
